# Initial kernel scaffold; baseline (speedup 1.0000x reference)
import jax
import jax.numpy as jnp
from jax.experimental import pallas as pl


def kernel(x, adj_indices, adj_values, W):
    raise NotImplementedError("write your pallas kernel here")


# speedup vs baseline: 2.6233x; 2.6233x over previous
import jax
import jax.numpy as jnp
from jax import lax
from jax.experimental import pallas as pl
from jax.experimental.pallas import tpu as pltpu
from jax.experimental.pallas import tpu_sc as plsc

N = 10000
E = 160000
F = 256
NQ = 4
Q = F // NQ
NC = 2
NS = 16
LANES = 16
K = 80
EPT = E // NS
NCHUNK = EPT // K
RCH = 200
NRC = N // RCH


def _mm_body(x_ref, w_ref, o_ref):
    o_ref[...] = jnp.dot(x_ref[...], w_ref[0],
                         preferred_element_type=jnp.float32)


def _matmul(xf, Wq):
    bm = 1000
    nb = N // bm
    return pl.pallas_call(
        _mm_body,
        grid=(NQ, nb),
        in_specs=[
            pl.BlockSpec((bm, F), lambda q, i: (i, 0)),
            pl.BlockSpec((1, F, Q), lambda q, i: (q, 0, 0)),
        ],
        out_specs=pl.BlockSpec((bm, Q), lambda q, i: (q * nb + i, 0)),
        out_shape=jax.ShapeDtypeStruct((NQ * N, Q), jnp.float32),
    )(xf, Wq)


def _sc_body(hidden, rows3, cols3, vals3, out_hbm,
             cols_v, rows_v, vals_v, gbuf, obuf, acc):
    c = lax.axis_index("c")
    s = lax.axis_index("s")

    pltpu.sync_copy(rows3.at[s], rows_v)
    pltpu.sync_copy(cols3.at[s], cols_v)
    pltpu.sync_copy(vals3.at[s], vals_v)

    def _add_off(off):
        offv = jnp.full((LANES,), off, dtype=jnp.int32)

        def _body(j, carry):
            for t in range(K // LANES):
                sl = pl.ds(t * LANES, LANES)
                cols_v[j, sl] = cols_v[j, sl] + offv
            return carry

        lax.fori_loop(0, NCHUNK, _body, 0)

    _add_off(2 * c * N)

    zero = jnp.zeros((LANES,), jnp.float32)

    def _zrow(i, carry):
        for t in range(Q // LANES):
            obuf[i, pl.ds(t * LANES, LANES)] = zero
        return carry

    def _relu_row(i, carry):
        for t in range(Q // LANES):
            sl = pl.ds(t * LANES, LANES)
            obuf[i, sl] = jnp.maximum(obuf[i, sl], 0.0)
        return carry

    def _chunk(j, carry):
        pltpu.sync_copy(hidden.at[cols_v.at[j]], gbuf)

        def _group(g, gcarry):
            vv = vals_v[0, pl.ds(j * K + g * LANES, LANES)]

            def _edge(i, ecarry):
                e = g * LANES + i
                val = vv.at[jnp.full((LANES,), i, dtype=jnp.int32)].get(
                    mode="promise_in_bounds")
                for t in range(Q // LANES):
                    sl = pl.ds(t * LANES, LANES)
                    gbuf[e, sl] = gbuf[e, sl] * val
                return ecarry

            return lax.fori_loop(0, LANES, _edge, gcarry)

        lax.fori_loop(0, K // LANES, _group, 0)
        pltpu.sync_copy(gbuf, acc.at[rows_v.at[j]], add=True)
        return carry

    for p in range(2):
        lax.fori_loop(0, RCH, _zrow, 0)
        for k in range((NRC + NS - 1) // NS):
            cid = s + k * NS

            @pl.when(cid < NRC)
            def _():
                r0 = pl.multiple_of(cid * RCH, 8)
                pltpu.sync_copy(obuf, acc.at[pl.ds(r0, RCH)])
        plsc.subcore_barrier()

        lax.fori_loop(0, NCHUNK, _chunk, 0)
        plsc.subcore_barrier()

        qq = 2 * c + p
        for k in range((NRC + NS - 1) // NS):
            cid = s + k * NS

            @pl.when(cid < NRC)
            def _():
                r0 = pl.multiple_of(cid * RCH, 8)
                pltpu.sync_copy(acc.at[pl.ds(r0, RCH)], obuf)
                lax.fori_loop(0, RCH, _relu_row, 0)
                o0 = pl.multiple_of(qq * N + r0, 8)
                pltpu.sync_copy(obuf, out_hbm.at[pl.ds(o0, RCH)])
        plsc.subcore_barrier()

        if p == 0:
            _add_off(N)


def _sc_aggregate(hidden, rows3, cols3, vals3):
    mesh = plsc.VectorSubcoreMesh(core_axis_name="c", subcore_axis_name="s")
    return pl.kernel(
        _sc_body,
        out_type=jax.ShapeDtypeStruct((NQ * N, Q), jnp.float32),
        mesh=mesh,
        compiler_params=pltpu.CompilerParams(use_tc_tiling_on_sc=False),
        scratch_types=[
            pltpu.VMEM((NCHUNK, K), jnp.int32),
            pltpu.VMEM((NCHUNK, K), jnp.int32),
            pltpu.VMEM((1, EPT), jnp.float32),
            pltpu.VMEM((K, Q), jnp.float32),
            pltpu.VMEM((RCH, Q), jnp.float32),
            pltpu.VMEM_SHARED((N, Q), jnp.float32),
        ],
    )(hidden, rows3, cols3, vals3)


def kernel(x, adj_indices, adj_values, W):
    xf = x.reshape(N, F)
    Wq = W.reshape(F, NQ, Q).transpose(1, 0, 2)
    hidden = _matmul(xf, Wq)
    rows3 = adj_indices[0].reshape(NS, NCHUNK, K)
    cols3 = adj_indices[1].reshape(NS, NCHUNK, K)
    vals3 = adj_values.reshape(NS, 1, EPT)
    o = _sc_aggregate(hidden, rows3, cols3, vals3)
    return o.reshape(NQ, N, Q).transpose(1, 0, 2).reshape(1, N, F)


# speedup vs baseline: 2.9389x; 1.1203x over previous
import jax
import jax.numpy as jnp
from jax import lax
from jax.experimental import pallas as pl
from jax.experimental.pallas import tpu as pltpu
from jax.experimental.pallas import tpu_sc as plsc

N = 10000
E = 160000
F = 256
NQ = 4
Q = F // NQ
NC = 2
NS = 16
LANES = 16
K = 80
EPT = E // NS
NCHUNK = EPT // K
RCH = 200
NRC = N // RCH


def _mm_body(x_ref, w_ref, o_ref):
    o_ref[...] = jnp.dot(x_ref[...], w_ref[0],
                         preferred_element_type=jnp.float32)


def _matmul(xf, Wq):
    bm = 1000
    nb = N // bm
    return pl.pallas_call(
        _mm_body,
        grid=(NQ, nb),
        in_specs=[
            pl.BlockSpec((bm, F), lambda q, i: (i, 0)),
            pl.BlockSpec((1, F, Q), lambda q, i: (q, 0, 0)),
        ],
        out_specs=pl.BlockSpec((bm, Q), lambda q, i: (q * nb + i, 0)),
        out_shape=jax.ShapeDtypeStruct((NQ * N, Q), jnp.float32),
    )(xf, Wq)


NB = 5
NITER = NCHUNK // NB


def _sc_body(hidden, rows3, cols3, vals3, out_hbm,
             cols_v, rows_v, vals_v, gbuf, obuf, acc, gsem, ssem):
    c = lax.axis_index("c")
    s = lax.axis_index("s")

    pltpu.sync_copy(rows3.at[s], rows_v)
    pltpu.sync_copy(cols3.at[s], cols_v)
    pltpu.sync_copy(vals3.at[s], vals_v)

    def _add_off(off):
        offv = jnp.full((LANES,), off, dtype=jnp.int32)

        def _body(j, carry):
            for t in range(K // LANES):
                sl = pl.ds(t * LANES, LANES)
                cols_v[j, sl] = cols_v[j, sl] + offv
            return carry

        lax.fori_loop(0, NCHUNK, _body, 0)

    _add_off(2 * c * N)

    zero = jnp.zeros((LANES,), jnp.float32)

    def _zrow(i, carry):
        for t in range(Q // LANES):
            obuf[i, pl.ds(t * LANES, LANES)] = zero
        return carry

    def _relu_row(i, carry):
        for t in range(Q // LANES):
            sl = pl.ds(t * LANES, LANES)
            obuf[i, sl] = jnp.maximum(obuf[i, sl], 0.0)
        return carry

    def _scale(j, b):
        def _group(g, gcarry):
            vv = vals_v[0, pl.ds(j * K + g * LANES, LANES)]
            for i in range(LANES):
                e = g * LANES + i
                val = vv.at[jnp.full((LANES,), i, dtype=jnp.int32)].get(
                    mode="promise_in_bounds")
                for t in range(Q // LANES):
                    sl = pl.ds(t * LANES, LANES)
                    gbuf[b, e, sl] = gbuf[b, e, sl] * val
            return gcarry

        lax.fori_loop(0, K // LANES, _group, 0)

    def _edge_pass():
        for b in range(NB):
            pltpu.async_copy(hidden.at[cols_v.at[b]], gbuf.at[b], gsem.at[b])

        def _iter(i, carry):
            m = i * NB
            for b in range(NB):
                j = m + b
                bp = (b + 2) % NB
                pltpu.make_async_copy(
                    hidden.at[cols_v.at[j]], gbuf.at[b], gsem.at[b]).wait()
                _scale(j, b)
                pltpu.async_copy(gbuf.at[b], acc.at[rows_v.at[j]],
                                 ssem.at[b], add=True)

                @pl.when(j >= 3)
                def _():
                    pltpu.make_async_copy(
                        gbuf.at[bp], acc.at[rows_v.at[j - 3]],
                        ssem.at[bp]).wait()

                @pl.when(jnp.logical_and(j >= 3, j < NCHUNK - 2))
                def _():
                    pltpu.async_copy(hidden.at[cols_v.at[j + 2]],
                                     gbuf.at[bp], gsem.at[bp])
            return carry

        lax.fori_loop(0, NITER, _iter, 0)
        for j in range(NCHUNK - 3, NCHUNK):
            pltpu.make_async_copy(
                gbuf.at[j % NB], acc.at[rows_v.at[j]],
                ssem.at[j % NB]).wait()

    for p in range(2):
        lax.fori_loop(0, RCH, _zrow, 0)
        for k in range((NRC + NS - 1) // NS):
            cid = s + k * NS

            @pl.when(cid < NRC)
            def _():
                r0 = pl.multiple_of(cid * RCH, 8)
                pltpu.sync_copy(obuf, acc.at[pl.ds(r0, RCH)])
        plsc.subcore_barrier()

        _edge_pass()
        plsc.subcore_barrier()

        qq = 2 * c + p
        for k in range((NRC + NS - 1) // NS):
            cid = s + k * NS

            @pl.when(cid < NRC)
            def _():
                r0 = pl.multiple_of(cid * RCH, 8)
                pltpu.sync_copy(acc.at[pl.ds(r0, RCH)], obuf)
                lax.fori_loop(0, RCH, _relu_row, 0)
                o0 = pl.multiple_of(qq * N + r0, 8)
                pltpu.sync_copy(obuf, out_hbm.at[pl.ds(o0, RCH)])
        plsc.subcore_barrier()

        if p == 0:
            _add_off(N)


def _sc_aggregate(hidden, rows3, cols3, vals3):
    mesh = plsc.VectorSubcoreMesh(core_axis_name="c", subcore_axis_name="s")
    return pl.kernel(
        _sc_body,
        out_type=jax.ShapeDtypeStruct((NQ * N, Q), jnp.float32),
        mesh=mesh,
        compiler_params=pltpu.CompilerParams(use_tc_tiling_on_sc=False),
        scratch_types=[
            pltpu.VMEM((NCHUNK, K), jnp.int32),
            pltpu.VMEM((NCHUNK, K), jnp.int32),
            pltpu.VMEM((1, EPT), jnp.float32),
            pltpu.VMEM((NB, K, Q), jnp.float32),
            pltpu.VMEM((RCH, Q), jnp.float32),
            pltpu.VMEM_SHARED((N, Q), jnp.float32),
            pltpu.SemaphoreType.DMA((NB,)),
            pltpu.SemaphoreType.DMA((NB,)),
        ],
    )(hidden, rows3, cols3, vals3)


def kernel(x, adj_indices, adj_values, W):
    xf = x.reshape(N, F)
    Wq = W.reshape(F, NQ, Q).transpose(1, 0, 2)
    hidden = _matmul(xf, Wq)
    rows3 = adj_indices[0].reshape(NS, NCHUNK, K)
    cols3 = adj_indices[1].reshape(NS, NCHUNK, K)
    vals3 = adj_values.reshape(NS, 1, EPT)
    o = _sc_aggregate(hidden, rows3, cols3, vals3)
    return o.reshape(NQ, N, Q).transpose(1, 0, 2).reshape(1, N, F)


# speedup vs baseline: 3.8145x; 1.2980x over previous
import numpy as np
import jax
import jax.numpy as jnp
from jax import lax
from jax.experimental import pallas as pl
from jax.experimental.pallas import tpu as pltpu
from jax.experimental.pallas import tpu_sc as plsc

N = 10000
E = 160000
F = 256
NQ = 4
Q = F // NQ
NC = 2
NS = 16
LANES = 16
K = 80
EPT = E // NS
NCHUNK = EPT // K
NB = 4
SB = 2
NITER = NCHUNK // NB
RCH = 40
NRC = N // RCH

_SIGMA = np.array([2 * l + 32 * (a // 2) + (a % 2)
                   for a in range(4) for l in range(16)])
_WPERM = np.concatenate([q * Q + np.argsort(_SIGMA) for q in range(NQ)])


def _mm_body(x_ref, w_ref, o_ref):
    o_ref[...] = jnp.dot(x_ref[...], w_ref[...],
                         preferred_element_type=jnp.float32
                         ).astype(jnp.bfloat16)


def _matmul(xf, Wp):
    bm = 2000
    nb = N // bm
    return pl.pallas_call(
        _mm_body,
        grid=(nb,),
        in_specs=[
            pl.BlockSpec((bm, F), lambda i: (i, 0)),
            pl.BlockSpec((F, F), lambda i: (0, 0)),
        ],
        out_specs=pl.BlockSpec((bm, F), lambda i: (i, 0)),
        out_shape=jax.ShapeDtypeStruct((N, F), jnp.bfloat16),
    )(xf, Wp)


def _sc_body(hidden, rows3, cols3, vals3, out_hbm,
             cols_v, rows_v, vals_v, gbuf, sbuf, obuf, hq, acc, gsem, ssem):
    c = lax.axis_index("c")
    s = lax.axis_index("s")

    pltpu.sync_copy(rows3.at[s], rows_v)
    pltpu.sync_copy(cols3.at[s], cols_v)
    pltpu.sync_copy(vals3.at[s], vals_v)

    zero = jnp.zeros((LANES,), jnp.float32)

    def _zrow(i, carry):
        for t in range(Q // LANES):
            obuf[i, pl.ds(t * LANES, LANES)] = zero
        return carry

    def _relu_row(i, carry):
        for t in range(Q // LANES):
            sl = pl.ds(t * LANES, LANES)
            obuf[i, sl] = jnp.maximum(obuf[i, sl], 0.0)
        return carry

    def _scale(j, b, bs):
        def _group(g, gcarry):
            vv = vals_v[0, pl.ds(j * K + g * LANES, LANES)]
            for i in range(LANES):
                e = g * LANES + i
                val = vv.at[jnp.full((LANES,), i, dtype=jnp.int32)].get(
                    mode="promise_in_bounds")
                for t in range(2):
                    lo, hi = plsc.unpack(
                        gbuf[b, e, pl.ds(32 * t, 32)],
                        format=plsc.PackFormat.INTERLEAVED)
                    sbuf[bs, e, pl.ds(32 * t, LANES)] = lo * val
                    sbuf[bs, e, pl.ds(32 * t + LANES, LANES)] = hi * val
            return gcarry

        lax.fori_loop(0, K // LANES, _group, 0)

    def _slot(j, b, bs):
        bp = (b + NB - 1) % NB
        pltpu.make_async_copy(
            hq.at[cols_v.at[j]], gbuf.at[b], gsem.at[b]).wait()

        @pl.when(j >= SB)
        def _():
            pltpu.make_async_copy(
                sbuf.at[bs], acc.at[rows_v.at[j - SB]], ssem.at[bs]).wait()

        _scale(j, b, bs)
        pltpu.async_copy(sbuf.at[bs], acc.at[rows_v.at[j]],
                         ssem.at[bs], add=True)

        @pl.when(jnp.logical_and(j >= 1, j <= NCHUNK - NB))
        def _():
            pltpu.async_copy(hq.at[cols_v.at[j + NB - 1]],
                             gbuf.at[bp], gsem.at[bp])

    for p in range(2):
        qq = 2 * c + p

        lax.fori_loop(0, RCH, _zrow, 0)
        for k in range((NRC + NS - 1) // NS):
            cid = s + k * NS

            @pl.when(cid < NRC)
            def _():
                r0 = pl.multiple_of(cid * RCH, 8)
                co = pl.multiple_of(qq * Q, 8)
                pltpu.sync_copy(obuf, acc.at[pl.ds(r0, RCH)])
                pltpu.sync_copy(hidden.at[pl.ds(r0, RCH), pl.ds(co, Q)],
                                hq.at[pl.ds(r0, RCH)])
        plsc.subcore_barrier()

        for b in range(NB):
            pltpu.async_copy(hq.at[cols_v.at[b]], gbuf.at[b], gsem.at[b])

        def _iter(i, carry):
            m = i * NB
            for b in range(NB):
                _slot(m + b, b, b % SB)
            return carry

        lax.fori_loop(0, NITER, _iter, 0)
        for j in range(NITER * NB, NCHUNK):
            _slot(j, j % NB, j % SB)
        for j in range(NCHUNK - SB, NCHUNK):
            pltpu.make_async_copy(
                sbuf.at[j % SB], acc.at[rows_v.at[j]], ssem.at[j % SB]).wait()
        plsc.subcore_barrier()

        for k in range((NRC + NS - 1) // NS):
            cid = s + k * NS

            @pl.when(cid < NRC)
            def _():
                r0 = pl.multiple_of(cid * RCH, 8)
                co = pl.multiple_of(qq * Q, 8)
                pltpu.sync_copy(acc.at[pl.ds(r0, RCH)], obuf)
                lax.fori_loop(0, RCH, _relu_row, 0)
                pltpu.sync_copy(obuf,
                                out_hbm.at[pl.ds(r0, RCH), pl.ds(co, Q)])
        plsc.subcore_barrier()


def _sc_aggregate(hidden, rows3, cols3, vals3):
    mesh = plsc.VectorSubcoreMesh(core_axis_name="c", subcore_axis_name="s")
    return pl.kernel(
        _sc_body,
        out_type=jax.ShapeDtypeStruct((N, F), jnp.float32),
        mesh=mesh,
        compiler_params=pltpu.CompilerParams(use_tc_tiling_on_sc=False,
                                             needs_layout_passes=False),
        scratch_types=[
            pltpu.VMEM((NCHUNK, K), jnp.int32),
            pltpu.VMEM((NCHUNK, K), jnp.int32),
            pltpu.VMEM((1, EPT), jnp.float32),
            pltpu.VMEM((NB, K, Q), jnp.bfloat16),
            pltpu.VMEM((SB, K, Q), jnp.float32),
            pltpu.VMEM((RCH, Q), jnp.float32),
            pltpu.VMEM_SHARED((N, Q), jnp.bfloat16),
            pltpu.VMEM_SHARED((N, Q), jnp.float32),
            pltpu.SemaphoreType.DMA((NB,)),
            pltpu.SemaphoreType.DMA((SB,)),
        ],
    )(hidden, rows3, cols3, vals3)


def kernel(x, adj_indices, adj_values, W):
    xf = x.reshape(N, F)
    hidden = _matmul(xf, W[:, _WPERM])
    rows3 = adj_indices[0].reshape(NS, NCHUNK, K)
    cols3 = adj_indices[1].reshape(NS, NCHUNK, K)
    vals3 = adj_values.reshape(NS, 1, EPT)
    o = _sc_aggregate(hidden, rows3, cols3, vals3)
    return o.reshape(1, N, F)
